# Initial kernel scaffold; baseline (speedup 1.0000x reference)
import jax
import jax.numpy as jnp
from jax.experimental import pallas as pl


def kernel(indices, embedding):
    raise NotImplementedError("write your pallas kernel here")


# speedup vs baseline: 2.9607x; 2.9607x over previous
import functools

import jax
import jax.numpy as jnp
from jax import lax
from jax.experimental import pallas as pl
from jax.experimental.pallas import tpu as pltpu
from jax.experimental.pallas import tpu_sc as plsc

NUM_EMBEDDINGS = 100000
EMBEDDING_DIM = 128
BATCH = 4096
HIST_LEN = 50
TOTAL = BATCH * HIST_LEN

_INFO = plsc.get_sparse_core_info()
NUM_CORES = _INFO.num_cores
NUM_SUBCORES = _INFO.num_subcores
NW = NUM_CORES * NUM_SUBCORES

CHUNK = 128
PER_W = TOTAL // NW
NCHUNK = PER_W // CHUNK

_mesh = plsc.VectorSubcoreMesh(core_axis_name="c", subcore_axis_name="s")


@functools.partial(
    pl.kernel,
    mesh=_mesh,
    out_type=jax.ShapeDtypeStruct((TOTAL, EMBEDDING_DIM), jnp.float32),
    scratch_types=[
        pltpu.VMEM((NCHUNK, CHUNK), jnp.int32),
        pltpu.VMEM((CHUNK, EMBEDDING_DIM), jnp.float32),
        pltpu.SemaphoreType.DMA,
    ],
)
def _gather_kernel(idx_hbm, table_hbm, out_hbm, idx_v, rows_v, gsem):
    wid = lax.axis_index("s") * NUM_CORES + lax.axis_index("c")
    pltpu.sync_copy(idx_hbm.at[wid], idx_v)

    def body(c, _):
        pltpu.async_copy(table_hbm.at[idx_v.at[c]], rows_v, gsem).wait()
        pltpu.sync_copy(rows_v, out_hbm.at[pl.ds((wid * NCHUNK + c) * CHUNK, CHUNK)])
        return ()

    lax.fori_loop(0, NCHUNK, body, (), unroll=False)


def kernel(indices, embedding):
    idx3d = indices.reshape(NW, NCHUNK, CHUNK).astype(jnp.int32)
    out = _gather_kernel(idx3d, embedding)
    return out.reshape(BATCH, HIST_LEN, EMBEDDING_DIM)


# speedup vs baseline: 3.3061x; 1.1167x over previous
import functools

import jax
import jax.numpy as jnp
from jax import lax
from jax.experimental import pallas as pl
from jax.experimental.pallas import tpu as pltpu
from jax.experimental.pallas import tpu_sc as plsc

NUM_EMBEDDINGS = 100000
EMBEDDING_DIM = 128
BATCH = 4096
HIST_LEN = 50
TOTAL = BATCH * HIST_LEN

_INFO = plsc.get_sparse_core_info()
NUM_CORES = _INFO.num_cores
NUM_SUBCORES = _INFO.num_subcores
NW = NUM_CORES * NUM_SUBCORES

CHUNK = 128
PER_W = TOTAL // NW
NCHUNK = PER_W // CHUNK
NBUF = 5

_mesh = plsc.VectorSubcoreMesh(core_axis_name="c", subcore_axis_name="s")


@functools.partial(
    pl.kernel,
    mesh=_mesh,
    out_type=jax.ShapeDtypeStruct((TOTAL, EMBEDDING_DIM), jnp.float32),
    scratch_types=[
        pltpu.VMEM((NCHUNK, CHUNK), jnp.int32),
        pltpu.VMEM((NBUF, CHUNK, EMBEDDING_DIM), jnp.float32),
        pltpu.SemaphoreType.DMA((NBUF,)),
        pltpu.SemaphoreType.DMA((NBUF,)),
    ],
)
def _gather_kernel(idx_hbm, table_hbm, out_hbm, idx_v, rows_v, gsem, wsem):
    wid = lax.axis_index("s") * NUM_CORES + lax.axis_index("c")
    pltpu.sync_copy(idx_hbm.at[wid], idx_v)
    out_base = wid * NCHUNK

    def start_gather(c, b):
        pltpu.async_copy(table_hbm.at[idx_v.at[c]], rows_v.at[b], gsem.at[b])

    def wait_gather(c, b):
        pltpu.make_async_copy(table_hbm.at[idx_v.at[c]], rows_v.at[b], gsem.at[b]).wait()

    def start_write(c, b):
        pltpu.async_copy(
            rows_v.at[b], out_hbm.at[pl.ds((out_base + c) * CHUNK, CHUNK)], wsem.at[b]
        )

    def wait_write(c, b):
        pltpu.make_async_copy(
            rows_v.at[b], out_hbm.at[pl.ds((out_base + c) * CHUNK, CHUNK)], wsem.at[b]
        ).wait()

    for b in range(NBUF):
        start_gather(b, b)

    @pl.loop(0, NCHUNK - NBUF, step=NBUF)
    def _steady(o):
        for b in range(NBUF):
            wait_gather(o + b, b)
            start_write(o + b, b)
        for b in range(NBUF):
            wait_write(o + b, b)
            start_gather(o + NBUF + b, b)

    tail = NCHUNK - NBUF
    for b in range(NBUF):
        wait_gather(tail + b, b)
        start_write(tail + b, b)
    for b in range(NBUF):
        wait_write(tail + b, b)


def kernel(indices, embedding):
    idx3d = indices.reshape(NW, NCHUNK, CHUNK).astype(jnp.int32)
    out = _gather_kernel(idx3d, embedding)
    return out.reshape(BATCH, HIST_LEN, EMBEDDING_DIM)


# speedup vs baseline: 5.9193x; 1.7904x over previous
import functools

import jax
import jax.numpy as jnp
from jax import lax
from jax.experimental import pallas as pl
from jax.experimental.pallas import tpu as pltpu
from jax.experimental.pallas import tpu_sc as plsc

NUM_EMBEDDINGS = 100000
EMBEDDING_DIM = 128
BATCH = 4096
HIST_LEN = 50

_INFO = plsc.get_sparse_core_info()
NUM_CORES = _INFO.num_cores
NUM_SUBCORES = _INFO.num_subcores
NW = NUM_CORES * NUM_SUBCORES

BPW = BATCH // NW
NBUF = 8

_mesh = plsc.VectorSubcoreMesh(core_axis_name="c", subcore_axis_name="s")


@functools.partial(
    pl.kernel,
    mesh=_mesh,
    out_type=jax.ShapeDtypeStruct((BATCH, HIST_LEN, EMBEDDING_DIM), jnp.float32),
    scratch_types=[
        pltpu.VMEM((BPW, HIST_LEN), jnp.int32),
        pltpu.VMEM((NBUF, HIST_LEN, EMBEDDING_DIM), jnp.float32),
        pltpu.SemaphoreType.DMA((NBUF,)),
        pltpu.SemaphoreType.DMA((NBUF,)),
    ],
)
def _gather_kernel(idx_hbm, table_hbm, out_hbm, idx_v, rows_v, gsem, wsem):
    wid = lax.axis_index("s") * NUM_CORES + lax.axis_index("c")
    base = wid * BPW
    pltpu.sync_copy(idx_hbm.at[pl.ds(base, BPW)], idx_v)

    def start_gather(j, b):
        pltpu.async_copy(table_hbm.at[idx_v.at[j]], rows_v.at[b], gsem.at[b])

    def wait_gather(j, b):
        pltpu.make_async_copy(table_hbm.at[idx_v.at[j]], rows_v.at[b], gsem.at[b]).wait()

    def start_write(j, b):
        pltpu.async_copy(rows_v.at[b], out_hbm.at[base + j], wsem.at[b])

    def wait_write(j, b):
        pltpu.make_async_copy(rows_v.at[b], out_hbm.at[base + j], wsem.at[b]).wait()

    for b in range(NBUF):
        start_gather(b, b)

    @pl.loop(0, BPW - NBUF, step=NBUF)
    def _steady(o):
        for b in range(NBUF):
            wait_gather(o + b, b)
            start_write(o + b, b)
        for b in range(NBUF):
            wait_write(o + b, b)
            start_gather(o + NBUF + b, b)

    tail = BPW - NBUF
    for b in range(NBUF):
        wait_gather(tail + b, b)
        start_write(tail + b, b)
    for b in range(NBUF):
        wait_write(tail + b, b)


def kernel(indices, embedding):
    return _gather_kernel(indices.astype(jnp.int32), embedding)


# speedup vs baseline: 10.3638x; 1.7509x over previous
import functools

import jax
import jax.numpy as jnp
from jax import lax
from jax.experimental import pallas as pl
from jax.experimental.pallas import tpu as pltpu
from jax.experimental.pallas import tpu_sc as plsc

NUM_EMBEDDINGS = 100000
EMBEDDING_DIM = 128
BATCH = 4096
HIST_LEN = 50

_INFO = plsc.get_sparse_core_info()
NUM_CORES = _INFO.num_cores
NUM_SUBCORES = _INFO.num_subcores
NW = NUM_CORES * NUM_SUBCORES

BCHUNK = BATCH // NW
NBUF = 5

_mesh = plsc.VectorSubcoreMesh(core_axis_name="c", subcore_axis_name="s")


@functools.partial(
    pl.kernel,
    mesh=_mesh,
    out_type=jax.ShapeDtypeStruct((HIST_LEN, BATCH, EMBEDDING_DIM), jnp.float32),
    scratch_types=[
        pltpu.VMEM((HIST_LEN, BCHUNK), jnp.int32),
        pltpu.VMEM((NBUF, BCHUNK, EMBEDDING_DIM), jnp.float32),
        pltpu.SemaphoreType.DMA((NBUF,)),
        pltpu.SemaphoreType.DMA((NBUF,)),
    ],
)
def _gather_kernel(idx_hbm, table_hbm, out_hbm, idx_v, rows_v, gsem, wsem):
    wid = lax.axis_index("s") * NUM_CORES + lax.axis_index("c")
    col = wid * BCHUNK
    pltpu.sync_copy(idx_hbm.at[:, pl.ds(col, BCHUNK)], idx_v)

    def start_gather(h, b):
        pltpu.async_copy(table_hbm.at[idx_v.at[h]], rows_v.at[b], gsem.at[b])

    def wait_gather(h, b):
        pltpu.make_async_copy(table_hbm.at[idx_v.at[h]], rows_v.at[b], gsem.at[b]).wait()

    def start_write(h, b):
        pltpu.async_copy(rows_v.at[b], out_hbm.at[h].at[pl.ds(col, BCHUNK)], wsem.at[b])

    def wait_write(h, b):
        pltpu.make_async_copy(
            rows_v.at[b], out_hbm.at[h].at[pl.ds(col, BCHUNK)], wsem.at[b]
        ).wait()

    for b in range(NBUF):
        start_gather(b, b)

    @pl.loop(0, HIST_LEN - NBUF, step=NBUF)
    def _steady(o):
        for b in range(NBUF):
            wait_gather(o + b, b)
            start_write(o + b, b)
        for b in range(NBUF):
            wait_write(o + b, b)
            start_gather(o + NBUF + b, b)

    tail = HIST_LEN - NBUF
    for b in range(NBUF):
        wait_gather(tail + b, b)
        start_write(tail + b, b)
    for b in range(NBUF):
        wait_write(tail + b, b)


def kernel(indices, embedding):
    out_phys = _gather_kernel(indices.astype(jnp.int32).T, embedding)
    return out_phys.transpose(1, 0, 2)
